# baseline (speedup 1.0000x reference)
import jax
import jax.numpy as jnp
from jax.experimental import pallas as pl
from jax.experimental.pallas import tpu as pltpu

B, N, K = 2, 4096, 16
R1 = 256
R2 = 512
NB1 = N // R1
NB2 = N // R2
CNT_LSE = float(B * N * K)
CNT_PT = float(B * N)
EPS = 1e-5
BIG = 3.0e38


def _mm(a, b):
    return jax.lax.dot_general(
        a.astype(jnp.bfloat16), b.astype(jnp.bfloat16),
        (((a.ndim - 1,), (0,)), ((), ())),
        preferred_element_type=jnp.float32)


def _mm_exact(a, b):
    return jax.lax.dot_general(a, b, (((a.ndim - 1,), (0,)), ((), ())),
                               preferred_element_type=jnp.float32,
                               precision=jax.lax.Precision.HIGHEST)


def _bn_affine(sums_ref, g_ref, bt_ref, cnt):
    s = sums_ref[:, 0:1] / cnt
    q = sums_ref[:, 1:2] / cnt
    var = q - s * s
    scale = g_ref[...] * jax.lax.rsqrt(var + EPS)
    shift = bt_ref[...] - scale * s
    return scale, shift


def _rowstats(y):
    return jnp.concatenate(
        [jnp.sum(y, axis=1, keepdims=True),
         jnp.sum(y * y, axis=1, keepdims=True)], axis=1)


CW = 128
NC = N // CW
BIGC = 2**30


def _k1_body(ctf_ref, rows_ref, cpl_ref, w1_ref, b1_ref,
             w2_ref, b2_ref, y1_ref, y3_ref, s1_ref, s3_ref, dist_ref):
    bidx = pl.program_id(0)
    ridx = pl.program_id(1)

    @pl.when(jnp.logical_and(bidx == 0, ridx == 0))
    def _():
        s1_ref[...] = jnp.zeros_like(s1_ref)
        s3_ref[...] = jnp.zeros_like(s3_ref)

    ct = ctf_ref[0]
    rows = rows_ref[0]
    planes = cpl_ref[0]

    d2a = jnp.sum(ct * ct, axis=0, keepdims=True)
    g = _mm(rows, ct)
    score = d2a - 2.0 * g
    dist_ref[...] = score

    lane = jax.lax.broadcasted_iota(jnp.int32, (R1, CW), 1)

    score3 = score.reshape(R1, NC, CW)
    Mv = jnp.min(score3, axis=2)
    lane3 = jax.lax.broadcasted_iota(jnp.int32, (R1, NC, CW), 2)
    ml = jnp.min(jnp.where(score3 <= Mv[:, :, None], lane3, CW), axis=2)
    ciota = jax.lax.broadcasted_iota(jnp.int32, (R1, NC), 1)
    Mc = ml + ciota * CW

    def body(k, carry):
        Mv, Mc = carry
        vmin = jnp.min(Mv, axis=1, keepdims=True)
        colmin = jnp.min(jnp.where(Mv <= vmin, Mc, BIGC),
                         axis=1, keepdims=True)
        cstar = jnp.right_shift(colmin, 7)
        lstar = colmin - (cstar << 7)
        ohb = ciota == cstar
        ohf = ohb.astype(jnp.float32)

        vals = dist_ref[:, 0:CW] * ohf[:, 0:1]
        for c in range(1, NC):
            vals = vals + dist_ref[:, c * CW:(c + 1) * CW] * ohf[:, c:c + 1]

        colv = lane + (cstar << 7)
        extracted = jnp.logical_or(
            vals < vmin, jnp.logical_and(vals == vmin, colv <= colmin))
        masked = jnp.where(extracted, BIG, vals)
        newv = jnp.min(masked, axis=1, keepdims=True)
        newl = jnp.min(jnp.where(masked <= newv, lane, CW),
                       axis=1, keepdims=True)
        newc = newl + (cstar << 7)
        Mv = jnp.where(ohb, newv, Mv)
        Mc = jnp.where(ohb, newc, Mc)

        sel = _mm_exact(ohf, planes)
        lm = (lane == lstar).astype(jnp.float32)
        nbx = jnp.sum(sel[:, 0:CW] * lm, axis=1, keepdims=True)
        nby = jnp.sum(sel[:, CW:2 * CW] * lm, axis=1, keepdims=True)
        nbz = jnp.sum(sel[:, 2 * CW:3 * CW] * lm, axis=1, keepdims=True)
        nbT = jnp.concatenate([nbx, nby, nbz], axis=1)
        diffT = rows - nbT
        dkT = jnp.sqrt(jnp.sum(diffT * diffT, axis=1, keepdims=True) + 1e-12)
        concatT = jnp.concatenate([rows, nbT, diffT, dkT], axis=1)
        concat = jnp.transpose(concatT)
        y1k = _mm(w1_ref[...], concat) + b1_ref[...]
        y3k = _mm(w2_ref[...], concat) + b2_ref[...]
        y1_ref[0, :, pl.ds(k, 1), :] = y1k[:, None, :]
        y3_ref[0, :, pl.ds(k, 1), :] = y3k[:, None, :]
        s1_ref[...] += _rowstats(y1k)
        s3_ref[...] += _rowstats(y3k)
        return Mv, Mc

    jax.lax.fori_loop(0, K, body, (Mv, Mc))


def _att_pool(x32, xf, c1_ref, c2_ref, s_ref, mw_ref, mb_ref):
    avg32 = jnp.mean(x32, axis=1)
    max32 = jnp.max(x32, axis=1)
    avg64 = jnp.concatenate([avg32, xf], axis=0)
    max64 = jnp.concatenate([max32, xf], axis=0)

    def cfc(t):
        h = jnp.maximum(_mm(c1_ref[...], t), 0.0)
        return _mm(c2_ref[...], h)

    ch = jax.nn.sigmoid(cfc(avg64) + cfc(max64))
    ch32 = ch[0:32][:, None, :]
    x32a = x32 * ch32
    xfa = xf * ch[32:64]

    a = (jnp.sum(x32a, axis=0) + jnp.sum(xfa, axis=0, keepdims=True)) / 64.0
    m = jnp.maximum(jnp.max(x32a, axis=0),
                    jnp.max(xfa, axis=0, keepdims=True))
    s00 = s_ref[0:1, 0:1]
    s01 = s_ref[0:1, 1:2]
    sp = jax.nn.sigmoid(s00 * a + s01 * m)
    fs32 = jnp.sum(x32a * sp[None, :, :], axis=1)
    fsf = xfa * jnp.sum(sp, axis=0, keepdims=True)
    fs = jnp.concatenate([fs32, fsf], axis=0)
    return _mm(mw_ref[...], fs) + mb_ref[...]


def _k2_body(y1_ref, s1_ref, f_ref, m1w_ref, m1b_ref, g1_ref, bt1_ref,
             c1_ref, c2_ref, s_ref, mw_ref, mb_ref, scw_ref, scb_ref,
             y2_ref, s2_ref, ssc_ref):
    bidx = pl.program_id(0)
    ridx = pl.program_id(1)

    @pl.when(jnp.logical_and(bidx == 0, ridx == 0))
    def _():
        s2_ref[...] = jnp.zeros_like(s2_ref)
        ssc_ref[...] = jnp.zeros_like(ssc_ref)

    scale, shift = _bn_affine(s1_ref, g1_ref, bt1_ref, CNT_LSE)
    x32 = jnp.maximum(scale[:, :, None] * y1_ref[0] + shift[:, :, None], 0.0)

    f = f_ref[0]
    xf0 = _mm(m1w_ref[...], f) + m1b_ref[...]
    xf = jnp.where(xf0 >= 0.0, xf0, 0.2 * xf0)

    y2 = _att_pool(x32, xf, c1_ref, c2_ref, s_ref, mw_ref, mb_ref)
    y2_ref[0] = y2
    s2_ref[...] += _rowstats(y2)

    shc = _mm(scw_ref[...], f) + scb_ref[...]
    ssc_ref[...] += _rowstats(shc)


def _k3_body(y3_ref, s3_ref, y2_ref, s2_ref, mg1_ref, mbt1_ref,
             g2_ref, bt2_ref, c1_ref, c2_ref, s_ref, mw_ref, mb_ref,
             y4_ref, s4_ref):
    bidx = pl.program_id(0)
    ridx = pl.program_id(1)

    @pl.when(jnp.logical_and(bidx == 0, ridx == 0))
    def _():
        s4_ref[...] = jnp.zeros_like(s4_ref)

    scale2, shift2 = _bn_affine(s2_ref, mg1_ref, mbt1_ref, CNT_PT)
    x2 = jnp.maximum(scale2 * y2_ref[0] + shift2, 0.0)

    scale3, shift3 = _bn_affine(s3_ref, g2_ref, bt2_ref, CNT_LSE)
    x32 = jnp.maximum(scale3[:, :, None] * y3_ref[0] + shift3[:, :, None],
                      0.0)

    y4 = _att_pool(x32, x2, c1_ref, c2_ref, s_ref, mw_ref, mb_ref)
    y4_ref[0] = y4
    s4_ref[...] += _rowstats(y4)


def _k4_body(y4_ref, s4_ref, mg2_ref, mbt2_ref, m2w_ref, m2b_ref,
             f_ref, scw_ref, scb_ref, ssc_ref, scg_ref, scbt_ref, o_ref):
    scale4, shift4 = _bn_affine(s4_ref, mg2_ref, mbt2_ref, CNT_PT)
    x3 = jnp.maximum(scale4 * y4_ref[0] + shift4, 0.0)
    main = _mm(m2w_ref[...], x3) + m2b_ref[...]

    f = f_ref[0]
    shc = _mm(scw_ref[...], f) + scb_ref[...]
    scs, scsh = _bn_affine(ssc_ref, scg_ref, scbt_ref, CNT_PT)
    o = main + (scs * shc + scsh)
    o_ref[0] = jnp.where(o >= 0.0, o, 0.01 * o)


def kernel(coords, features, mlp1_w, mlp1_b, lse1_w, lse1_b, lse1_g,
           lse1_bt, lse2_w, lse2_b, lse2_g, lse2_bt, p1_c1, p1_c2, p1_s,
           p1_mw, p1_mb, p1_mg, p1_mbt, p2_c1, p2_c2, p2_s, p2_mw, p2_mb,
           p2_mg, p2_mbt, mlp2_w, mlp2_b, sc_w, sc_b, sc_g, sc_bt):
    f32 = jnp.float32
    coordsT = jnp.transpose(coords, (0, 2, 1))
    coordsP = jnp.transpose(coords.reshape(B, NC, CW, 3),
                            (0, 1, 3, 2)).reshape(B, NC, 3 * CW)
    fR = features[:, :, :, 0]
    col = lambda v: v.reshape(-1, 1)

    y1, y3, s1, s3 = pl.pallas_call(
        _k1_body,
        grid=(B, NB1),
        in_specs=[
            pl.BlockSpec((1, 3, N), lambda b, r: (b, 0, 0)),
            pl.BlockSpec((1, R1, 3), lambda b, r: (b, r, 0)),
            pl.BlockSpec((1, NC, 3 * CW), lambda b, r: (b, 0, 0)),
            pl.BlockSpec((32, 10), lambda b, r: (0, 0)),
            pl.BlockSpec((32, 1), lambda b, r: (0, 0)),
            pl.BlockSpec((32, 10), lambda b, r: (0, 0)),
            pl.BlockSpec((32, 1), lambda b, r: (0, 0)),
        ],
        out_specs=[
            pl.BlockSpec((1, 32, K, R1), lambda b, r: (b, 0, 0, r)),
            pl.BlockSpec((1, 32, K, R1), lambda b, r: (b, 0, 0, r)),
            pl.BlockSpec((32, 2), lambda b, r: (0, 0)),
            pl.BlockSpec((32, 2), lambda b, r: (0, 0)),
        ],
        out_shape=[
            jax.ShapeDtypeStruct((B, 32, K, N), f32),
            jax.ShapeDtypeStruct((B, 32, K, N), f32),
            jax.ShapeDtypeStruct((32, 2), f32),
            jax.ShapeDtypeStruct((32, 2), f32),
        ],
        scratch_shapes=[pltpu.VMEM((R1, N), f32)],
    )(coordsT, coords, coordsP, lse1_w, col(lse1_b), lse2_w, col(lse2_b))

    y2, s2, ssc = pl.pallas_call(
        _k2_body,
        grid=(B, NB2),
        in_specs=[
            pl.BlockSpec((1, 32, K, R2), lambda b, r: (b, 0, 0, r)),
            pl.BlockSpec((32, 2), lambda b, r: (0, 0)),
            pl.BlockSpec((1, 8, R2), lambda b, r: (b, 0, r)),
            pl.BlockSpec((32, 8), lambda b, r: (0, 0)),
            pl.BlockSpec((32, 1), lambda b, r: (0, 0)),
            pl.BlockSpec((32, 1), lambda b, r: (0, 0)),
            pl.BlockSpec((32, 1), lambda b, r: (0, 0)),
            pl.BlockSpec((8, 64), lambda b, r: (0, 0)),
            pl.BlockSpec((64, 8), lambda b, r: (0, 0)),
            pl.BlockSpec((1, 2), lambda b, r: (0, 0)),
            pl.BlockSpec((32, 64), lambda b, r: (0, 0)),
            pl.BlockSpec((32, 1), lambda b, r: (0, 0)),
            pl.BlockSpec((128, 8), lambda b, r: (0, 0)),
            pl.BlockSpec((128, 1), lambda b, r: (0, 0)),
        ],
        out_specs=[
            pl.BlockSpec((1, 32, R2), lambda b, r: (b, 0, r)),
            pl.BlockSpec((32, 2), lambda b, r: (0, 0)),
            pl.BlockSpec((128, 2), lambda b, r: (0, 0)),
        ],
        out_shape=[
            jax.ShapeDtypeStruct((B, 32, N), f32),
            jax.ShapeDtypeStruct((32, 2), f32),
            jax.ShapeDtypeStruct((128, 2), f32),
        ],
    )(y1, s1, fR, mlp1_w, col(mlp1_b), col(lse1_g), col(lse1_bt),
      p1_c1, p1_c2, p1_s, p1_mw, col(p1_mb), sc_w, col(sc_b))

    y4, s4 = pl.pallas_call(
        _k3_body,
        grid=(B, NB2),
        in_specs=[
            pl.BlockSpec((1, 32, K, R2), lambda b, r: (b, 0, 0, r)),
            pl.BlockSpec((32, 2), lambda b, r: (0, 0)),
            pl.BlockSpec((1, 32, R2), lambda b, r: (b, 0, r)),
            pl.BlockSpec((32, 2), lambda b, r: (0, 0)),
            pl.BlockSpec((32, 1), lambda b, r: (0, 0)),
            pl.BlockSpec((32, 1), lambda b, r: (0, 0)),
            pl.BlockSpec((32, 1), lambda b, r: (0, 0)),
            pl.BlockSpec((32, 1), lambda b, r: (0, 0)),
            pl.BlockSpec((8, 64), lambda b, r: (0, 0)),
            pl.BlockSpec((64, 8), lambda b, r: (0, 0)),
            pl.BlockSpec((1, 2), lambda b, r: (0, 0)),
            pl.BlockSpec((64, 64), lambda b, r: (0, 0)),
            pl.BlockSpec((64, 1), lambda b, r: (0, 0)),
        ],
        out_specs=[
            pl.BlockSpec((1, 64, R2), lambda b, r: (b, 0, r)),
            pl.BlockSpec((64, 2), lambda b, r: (0, 0)),
        ],
        out_shape=[
            jax.ShapeDtypeStruct((B, 64, N), f32),
            jax.ShapeDtypeStruct((64, 2), f32),
        ],
    )(y3, s3, y2, s2, col(p1_mg), col(p1_mbt), col(lse2_g), col(lse2_bt),
      p2_c1, p2_c2, p2_s, p2_mw, col(p2_mb))

    out = pl.pallas_call(
        _k4_body,
        grid=(B, NB2),
        in_specs=[
            pl.BlockSpec((1, 64, R2), lambda b, r: (b, 0, r)),
            pl.BlockSpec((64, 2), lambda b, r: (0, 0)),
            pl.BlockSpec((64, 1), lambda b, r: (0, 0)),
            pl.BlockSpec((64, 1), lambda b, r: (0, 0)),
            pl.BlockSpec((128, 64), lambda b, r: (0, 0)),
            pl.BlockSpec((128, 1), lambda b, r: (0, 0)),
            pl.BlockSpec((1, 8, R2), lambda b, r: (b, 0, r)),
            pl.BlockSpec((128, 8), lambda b, r: (0, 0)),
            pl.BlockSpec((128, 1), lambda b, r: (0, 0)),
            pl.BlockSpec((128, 2), lambda b, r: (0, 0)),
            pl.BlockSpec((128, 1), lambda b, r: (0, 0)),
            pl.BlockSpec((128, 1), lambda b, r: (0, 0)),
        ],
        out_specs=[pl.BlockSpec((1, 128, R2), lambda b, r: (b, 0, r))],
        out_shape=[jax.ShapeDtypeStruct((B, 128, N), f32)],
    )(y4, s4, col(p2_mg), col(p2_mbt), mlp2_w, col(mlp2_b), fR,
      sc_w, col(sc_b), ssc, col(sc_g), col(sc_bt))[0]

    return out[:, :, :, None]


# speedup vs baseline: 1.2044x; 1.2044x over previous
import jax
import jax.numpy as jnp
from jax.experimental import pallas as pl
from jax.experimental.pallas import tpu as pltpu

B, N, K = 2, 4096, 16
R1 = 256
R2 = 512
NB1 = N // R1
NB2 = N // R2
CNT_LSE = float(B * N * K)
CNT_PT = float(B * N)
EPS = 1e-5
BIG = 3.0e38


def _mm(a, b):
    return jax.lax.dot_general(
        a.astype(jnp.bfloat16), b.astype(jnp.bfloat16),
        (((a.ndim - 1,), (0,)), ((), ())),
        preferred_element_type=jnp.float32)


def _mm_exact(a, b):
    return jax.lax.dot_general(a, b, (((a.ndim - 1,), (0,)), ((), ())),
                               preferred_element_type=jnp.float32,
                               precision=jax.lax.Precision.HIGHEST)


def _bn_affine(sums_ref, g_ref, bt_ref, cnt):
    s = sums_ref[:, 0:1] / cnt
    q = sums_ref[:, 1:2] / cnt
    var = q - s * s
    scale = g_ref[...] * jax.lax.rsqrt(var + EPS)
    shift = bt_ref[...] - scale * s
    return scale, shift


def _rowstats(y):
    return jnp.concatenate(
        [jnp.sum(y, axis=1, keepdims=True),
         jnp.sum(y * y, axis=1, keepdims=True)], axis=1)


CW = 128
NC = N // CW


def _k1_body(ctf_ref, rows_ref, cpl_ref, w1_ref, b1_ref,
             w2_ref, b2_ref, y1_ref, y3_ref, s1_ref, s3_ref, dist_ref):
    bidx = pl.program_id(0)
    ridx = pl.program_id(1)

    @pl.when(jnp.logical_and(bidx == 0, ridx == 0))
    def _():
        s1_ref[...] = jnp.zeros_like(s1_ref)
        s3_ref[...] = jnp.zeros_like(s3_ref)

    ct = ctf_ref[0]
    rows = rows_ref[0]
    planes = cpl_ref[0]

    d2a = jnp.sum(ct * ct, axis=0, keepdims=True)
    g = _mm(rows, ct)
    score = d2a - 2.0 * g
    dist_ref[...] = score

    laneF = jax.lax.broadcasted_iota(jnp.int32, (R1, CW), 1).astype(
        jnp.float32)
    ciotaF = jax.lax.broadcasted_iota(jnp.int32, (R1, NC), 1).astype(
        jnp.float32)

    Mv = jnp.concatenate(
        [jnp.min(score[:, c * CW:(c + 1) * CW], axis=1, keepdims=True)
         for c in range(NC)], axis=1)
    Ml = jnp.full((R1, NC), -1.0, jnp.float32)

    def body(k, carry):
        Mv, Ml = carry
        vmin = jnp.min(Mv, axis=1, keepdims=True)
        cstar = jnp.min(jnp.where(Mv <= vmin, ciotaF, float(NC)),
                        axis=1, keepdims=True)
        ohb = ciotaF == cstar
        ohf = ohb.astype(jnp.float32)

        vals = dist_ref[:, 0:CW] * ohf[:, 0:1]
        for c in range(1, NC):
            vals = vals + dist_ref[:, c * CW:(c + 1) * CW] * ohf[:, c:c + 1]

        lraw = jnp.min(jnp.where(ohb, Ml, float(CW)), axis=1, keepdims=True)
        lfresh = jnp.min(jnp.where(vals <= vmin, laneF, float(CW)),
                         axis=1, keepdims=True)
        lstar = jnp.where(lraw < 0.0, lfresh, lraw)

        extracted = jnp.logical_or(
            vals < vmin, jnp.logical_and(vals == vmin, laneF <= lstar))
        masked = jnp.where(extracted, BIG, vals)
        newv = jnp.min(masked, axis=1, keepdims=True)
        newl = jnp.min(jnp.where(masked <= newv, laneF, float(CW)),
                       axis=1, keepdims=True)
        Mv = jnp.where(ohb, newv, Mv)
        Ml = jnp.where(ohb, newl, Ml)

        sel = _mm_exact(ohf, planes)
        lm = (laneF == lstar).astype(jnp.float32)
        nbx = jnp.sum(sel[:, 0:CW] * lm, axis=1, keepdims=True)
        nby = jnp.sum(sel[:, CW:2 * CW] * lm, axis=1, keepdims=True)
        nbz = jnp.sum(sel[:, 2 * CW:3 * CW] * lm, axis=1, keepdims=True)
        nbT = jnp.concatenate([nbx, nby, nbz], axis=1)
        diffT = rows - nbT
        dkT = jnp.sqrt(jnp.sum(diffT * diffT, axis=1, keepdims=True) + 1e-12)
        concatT = jnp.concatenate([rows, nbT, diffT, dkT], axis=1)
        concat = jnp.transpose(concatT)
        y1k = _mm(w1_ref[...], concat) + b1_ref[...]
        y3k = _mm(w2_ref[...], concat) + b2_ref[...]
        y1_ref[0, :, pl.ds(k, 1), :] = y1k[:, None, :]
        y3_ref[0, :, pl.ds(k, 1), :] = y3k[:, None, :]
        s1_ref[...] += _rowstats(y1k)
        s3_ref[...] += _rowstats(y3k)
        return Mv, Ml

    jax.lax.fori_loop(0, K, body, (Mv, Ml))


def _att_pool(x32, xf, c1_ref, c2_ref, s_ref, mw_ref, mb_ref):
    avg32 = jnp.mean(x32, axis=1)
    max32 = jnp.max(x32, axis=1)
    avg64 = jnp.concatenate([avg32, xf], axis=0)
    max64 = jnp.concatenate([max32, xf], axis=0)

    def cfc(t):
        h = jnp.maximum(_mm(c1_ref[...], t), 0.0)
        return _mm(c2_ref[...], h)

    ch = jax.nn.sigmoid(cfc(avg64) + cfc(max64))
    ch32 = ch[0:32][:, None, :]
    x32a = x32 * ch32
    xfa = xf * ch[32:64]

    a = (jnp.sum(x32a, axis=0) + jnp.sum(xfa, axis=0, keepdims=True)) / 64.0
    m = jnp.maximum(jnp.max(x32a, axis=0),
                    jnp.max(xfa, axis=0, keepdims=True))
    s00 = s_ref[0:1, 0:1]
    s01 = s_ref[0:1, 1:2]
    sp = jax.nn.sigmoid(s00 * a + s01 * m)
    fs32 = jnp.sum(x32a * sp[None, :, :], axis=1)
    fsf = xfa * jnp.sum(sp, axis=0, keepdims=True)
    fs = jnp.concatenate([fs32, fsf], axis=0)
    return _mm(mw_ref[...], fs) + mb_ref[...]


def _k2_body(y1_ref, s1_ref, f_ref, m1w_ref, m1b_ref, g1_ref, bt1_ref,
             c1_ref, c2_ref, s_ref, mw_ref, mb_ref, scw_ref, scb_ref,
             y2_ref, s2_ref, ssc_ref):
    bidx = pl.program_id(0)
    ridx = pl.program_id(1)

    @pl.when(jnp.logical_and(bidx == 0, ridx == 0))
    def _():
        s2_ref[...] = jnp.zeros_like(s2_ref)
        ssc_ref[...] = jnp.zeros_like(ssc_ref)

    scale, shift = _bn_affine(s1_ref, g1_ref, bt1_ref, CNT_LSE)
    x32 = jnp.maximum(scale[:, :, None] * y1_ref[0] + shift[:, :, None], 0.0)

    f = f_ref[0]
    xf0 = _mm(m1w_ref[...], f) + m1b_ref[...]
    xf = jnp.where(xf0 >= 0.0, xf0, 0.2 * xf0)

    y2 = _att_pool(x32, xf, c1_ref, c2_ref, s_ref, mw_ref, mb_ref)
    y2_ref[0] = y2
    s2_ref[...] += _rowstats(y2)

    shc = _mm(scw_ref[...], f) + scb_ref[...]
    ssc_ref[...] += _rowstats(shc)


def _k3_body(y3_ref, s3_ref, y2_ref, s2_ref, mg1_ref, mbt1_ref,
             g2_ref, bt2_ref, c1_ref, c2_ref, s_ref, mw_ref, mb_ref,
             y4_ref, s4_ref):
    bidx = pl.program_id(0)
    ridx = pl.program_id(1)

    @pl.when(jnp.logical_and(bidx == 0, ridx == 0))
    def _():
        s4_ref[...] = jnp.zeros_like(s4_ref)

    scale2, shift2 = _bn_affine(s2_ref, mg1_ref, mbt1_ref, CNT_PT)
    x2 = jnp.maximum(scale2 * y2_ref[0] + shift2, 0.0)

    scale3, shift3 = _bn_affine(s3_ref, g2_ref, bt2_ref, CNT_LSE)
    x32 = jnp.maximum(scale3[:, :, None] * y3_ref[0] + shift3[:, :, None],
                      0.0)

    y4 = _att_pool(x32, x2, c1_ref, c2_ref, s_ref, mw_ref, mb_ref)
    y4_ref[0] = y4
    s4_ref[...] += _rowstats(y4)


def _k4_body(y4_ref, s4_ref, mg2_ref, mbt2_ref, m2w_ref, m2b_ref,
             f_ref, scw_ref, scb_ref, ssc_ref, scg_ref, scbt_ref, o_ref):
    scale4, shift4 = _bn_affine(s4_ref, mg2_ref, mbt2_ref, CNT_PT)
    x3 = jnp.maximum(scale4 * y4_ref[0] + shift4, 0.0)
    main = _mm(m2w_ref[...], x3) + m2b_ref[...]

    f = f_ref[0]
    shc = _mm(scw_ref[...], f) + scb_ref[...]
    scs, scsh = _bn_affine(ssc_ref, scg_ref, scbt_ref, CNT_PT)
    o = main + (scs * shc + scsh)
    o_ref[0] = jnp.where(o >= 0.0, o, 0.01 * o)


def kernel(coords, features, mlp1_w, mlp1_b, lse1_w, lse1_b, lse1_g,
           lse1_bt, lse2_w, lse2_b, lse2_g, lse2_bt, p1_c1, p1_c2, p1_s,
           p1_mw, p1_mb, p1_mg, p1_mbt, p2_c1, p2_c2, p2_s, p2_mw, p2_mb,
           p2_mg, p2_mbt, mlp2_w, mlp2_b, sc_w, sc_b, sc_g, sc_bt):
    f32 = jnp.float32
    coordsT = jnp.transpose(coords, (0, 2, 1))
    coordsP = jnp.transpose(coords.reshape(B, NC, CW, 3),
                            (0, 1, 3, 2)).reshape(B, NC, 3 * CW)
    fR = features[:, :, :, 0]
    col = lambda v: v.reshape(-1, 1)

    y1, y3, s1, s3 = pl.pallas_call(
        _k1_body,
        grid=(B, NB1),
        in_specs=[
            pl.BlockSpec((1, 3, N), lambda b, r: (b, 0, 0)),
            pl.BlockSpec((1, R1, 3), lambda b, r: (b, r, 0)),
            pl.BlockSpec((1, NC, 3 * CW), lambda b, r: (b, 0, 0)),
            pl.BlockSpec((32, 10), lambda b, r: (0, 0)),
            pl.BlockSpec((32, 1), lambda b, r: (0, 0)),
            pl.BlockSpec((32, 10), lambda b, r: (0, 0)),
            pl.BlockSpec((32, 1), lambda b, r: (0, 0)),
        ],
        out_specs=[
            pl.BlockSpec((1, 32, K, R1), lambda b, r: (b, 0, 0, r)),
            pl.BlockSpec((1, 32, K, R1), lambda b, r: (b, 0, 0, r)),
            pl.BlockSpec((32, 2), lambda b, r: (0, 0)),
            pl.BlockSpec((32, 2), lambda b, r: (0, 0)),
        ],
        out_shape=[
            jax.ShapeDtypeStruct((B, 32, K, N), f32),
            jax.ShapeDtypeStruct((B, 32, K, N), f32),
            jax.ShapeDtypeStruct((32, 2), f32),
            jax.ShapeDtypeStruct((32, 2), f32),
        ],
        scratch_shapes=[pltpu.VMEM((R1, N), f32)],
    )(coordsT, coords, coordsP, lse1_w, col(lse1_b), lse2_w, col(lse2_b))

    y2, s2, ssc = pl.pallas_call(
        _k2_body,
        grid=(B, NB2),
        in_specs=[
            pl.BlockSpec((1, 32, K, R2), lambda b, r: (b, 0, 0, r)),
            pl.BlockSpec((32, 2), lambda b, r: (0, 0)),
            pl.BlockSpec((1, 8, R2), lambda b, r: (b, 0, r)),
            pl.BlockSpec((32, 8), lambda b, r: (0, 0)),
            pl.BlockSpec((32, 1), lambda b, r: (0, 0)),
            pl.BlockSpec((32, 1), lambda b, r: (0, 0)),
            pl.BlockSpec((32, 1), lambda b, r: (0, 0)),
            pl.BlockSpec((8, 64), lambda b, r: (0, 0)),
            pl.BlockSpec((64, 8), lambda b, r: (0, 0)),
            pl.BlockSpec((1, 2), lambda b, r: (0, 0)),
            pl.BlockSpec((32, 64), lambda b, r: (0, 0)),
            pl.BlockSpec((32, 1), lambda b, r: (0, 0)),
            pl.BlockSpec((128, 8), lambda b, r: (0, 0)),
            pl.BlockSpec((128, 1), lambda b, r: (0, 0)),
        ],
        out_specs=[
            pl.BlockSpec((1, 32, R2), lambda b, r: (b, 0, r)),
            pl.BlockSpec((32, 2), lambda b, r: (0, 0)),
            pl.BlockSpec((128, 2), lambda b, r: (0, 0)),
        ],
        out_shape=[
            jax.ShapeDtypeStruct((B, 32, N), f32),
            jax.ShapeDtypeStruct((32, 2), f32),
            jax.ShapeDtypeStruct((128, 2), f32),
        ],
    )(y1, s1, fR, mlp1_w, col(mlp1_b), col(lse1_g), col(lse1_bt),
      p1_c1, p1_c2, p1_s, p1_mw, col(p1_mb), sc_w, col(sc_b))

    y4, s4 = pl.pallas_call(
        _k3_body,
        grid=(B, NB2),
        in_specs=[
            pl.BlockSpec((1, 32, K, R2), lambda b, r: (b, 0, 0, r)),
            pl.BlockSpec((32, 2), lambda b, r: (0, 0)),
            pl.BlockSpec((1, 32, R2), lambda b, r: (b, 0, r)),
            pl.BlockSpec((32, 2), lambda b, r: (0, 0)),
            pl.BlockSpec((32, 1), lambda b, r: (0, 0)),
            pl.BlockSpec((32, 1), lambda b, r: (0, 0)),
            pl.BlockSpec((32, 1), lambda b, r: (0, 0)),
            pl.BlockSpec((32, 1), lambda b, r: (0, 0)),
            pl.BlockSpec((8, 64), lambda b, r: (0, 0)),
            pl.BlockSpec((64, 8), lambda b, r: (0, 0)),
            pl.BlockSpec((1, 2), lambda b, r: (0, 0)),
            pl.BlockSpec((64, 64), lambda b, r: (0, 0)),
            pl.BlockSpec((64, 1), lambda b, r: (0, 0)),
        ],
        out_specs=[
            pl.BlockSpec((1, 64, R2), lambda b, r: (b, 0, r)),
            pl.BlockSpec((64, 2), lambda b, r: (0, 0)),
        ],
        out_shape=[
            jax.ShapeDtypeStruct((B, 64, N), f32),
            jax.ShapeDtypeStruct((64, 2), f32),
        ],
    )(y3, s3, y2, s2, col(p1_mg), col(p1_mbt), col(lse2_g), col(lse2_bt),
      p2_c1, p2_c2, p2_s, p2_mw, col(p2_mb))

    out = pl.pallas_call(
        _k4_body,
        grid=(B, NB2),
        in_specs=[
            pl.BlockSpec((1, 64, R2), lambda b, r: (b, 0, r)),
            pl.BlockSpec((64, 2), lambda b, r: (0, 0)),
            pl.BlockSpec((64, 1), lambda b, r: (0, 0)),
            pl.BlockSpec((64, 1), lambda b, r: (0, 0)),
            pl.BlockSpec((128, 64), lambda b, r: (0, 0)),
            pl.BlockSpec((128, 1), lambda b, r: (0, 0)),
            pl.BlockSpec((1, 8, R2), lambda b, r: (b, 0, r)),
            pl.BlockSpec((128, 8), lambda b, r: (0, 0)),
            pl.BlockSpec((128, 1), lambda b, r: (0, 0)),
            pl.BlockSpec((128, 2), lambda b, r: (0, 0)),
            pl.BlockSpec((128, 1), lambda b, r: (0, 0)),
            pl.BlockSpec((128, 1), lambda b, r: (0, 0)),
        ],
        out_specs=[pl.BlockSpec((1, 128, R2), lambda b, r: (b, 0, r))],
        out_shape=[jax.ShapeDtypeStruct((B, 128, N), f32)],
    )(y4, s4, col(p2_mg), col(p2_mbt), mlp2_w, col(mlp2_b), fR,
      sc_w, col(sc_b), ssc, col(sc_g), col(sc_bt))[0]

    return out[:, :, :, None]


# speedup vs baseline: 1.5115x; 1.2550x over previous
import jax
import jax.numpy as jnp
from jax.experimental import pallas as pl
from jax.experimental.pallas import tpu as pltpu

B, N, K = 2, 4096, 16
R1 = 512
R2 = 512
NB1 = N // R1
NB2 = N // R2
CNT_LSE = float(B * N * K)
CNT_PT = float(B * N)
EPS = 1e-5
BIG = 3.0e38


def _mm(a, b):
    return jax.lax.dot_general(
        a.astype(jnp.bfloat16), b.astype(jnp.bfloat16),
        (((a.ndim - 1,), (0,)), ((), ())),
        preferred_element_type=jnp.float32)


def _mm_exact(a, b):
    return jax.lax.dot_general(a, b, (((a.ndim - 1,), (0,)), ((), ())),
                               preferred_element_type=jnp.float32,
                               precision=jax.lax.Precision.HIGHEST)


def _bn_affine(sums_ref, g_ref, bt_ref, cnt):
    s = sums_ref[:, 0:1] / cnt
    q = sums_ref[:, 1:2] / cnt
    var = q - s * s
    scale = g_ref[...] * jax.lax.rsqrt(var + EPS)
    shift = bt_ref[...] - scale * s
    return scale, shift


def _rowstats(y):
    return jnp.concatenate(
        [jnp.sum(y, axis=1, keepdims=True),
         jnp.sum(y * y, axis=1, keepdims=True)], axis=1)


CW = 128
NC = N // CW


def _k1_body(ctf_ref, rows_ref, cpl_ref, w1_ref, b1_ref,
             w2_ref, b2_ref, y1_ref, y3_ref, s1_ref, s3_ref, dist_ref):
    bidx = pl.program_id(0)
    ridx = pl.program_id(1)

    @pl.when(jnp.logical_and(bidx == 0, ridx == 0))
    def _():
        s1_ref[...] = jnp.zeros_like(s1_ref)
        s3_ref[...] = jnp.zeros_like(s3_ref)

    ct = ctf_ref[0]
    rows = rows_ref[0]
    planes = cpl_ref[0]

    d2a = jnp.sum(ct * ct, axis=0, keepdims=True)
    g = _mm(rows, ct)
    score = d2a - 2.0 * g
    dist_ref[...] = score

    laneF = jax.lax.broadcasted_iota(jnp.int32, (R1, CW), 1).astype(
        jnp.float32)
    ciotaF = jax.lax.broadcasted_iota(jnp.int32, (R1, NC), 1).astype(
        jnp.float32)

    Mv = jnp.concatenate(
        [jnp.min(score[:, c * CW:(c + 1) * CW], axis=1, keepdims=True)
         for c in range(NC)], axis=1)
    Ml = jnp.full((R1, NC), -1.0, jnp.float32)

    def body(k, carry):
        Mv, Ml = carry
        vmin = jnp.min(Mv, axis=1, keepdims=True)
        cstar = jnp.min(jnp.where(Mv <= vmin, ciotaF, float(NC)),
                        axis=1, keepdims=True)
        ohb = ciotaF == cstar
        ohf = ohb.astype(jnp.float32)

        vals = dist_ref[:, 0:CW] * ohf[:, 0:1]
        for c in range(1, NC):
            vals = vals + dist_ref[:, c * CW:(c + 1) * CW] * ohf[:, c:c + 1]

        lraw = jnp.min(jnp.where(ohb, Ml, float(CW)), axis=1, keepdims=True)
        lfresh = jnp.min(jnp.where(vals <= vmin, laneF, float(CW)),
                         axis=1, keepdims=True)
        lstar = jnp.where(lraw < 0.0, lfresh, lraw)

        extracted = jnp.logical_or(
            vals < vmin, jnp.logical_and(vals == vmin, laneF <= lstar))
        masked = jnp.where(extracted, BIG, vals)
        newv = jnp.min(masked, axis=1, keepdims=True)
        newl = jnp.min(jnp.where(masked <= newv, laneF, float(CW)),
                       axis=1, keepdims=True)
        Mv = jnp.where(ohb, newv, Mv)
        Ml = jnp.where(ohb, newl, Ml)

        sel = _mm_exact(ohf, planes)
        lm = (laneF == lstar).astype(jnp.float32)
        nbx = jnp.sum(sel[:, 0:CW] * lm, axis=1, keepdims=True)
        nby = jnp.sum(sel[:, CW:2 * CW] * lm, axis=1, keepdims=True)
        nbz = jnp.sum(sel[:, 2 * CW:3 * CW] * lm, axis=1, keepdims=True)
        nbT = jnp.concatenate([nbx, nby, nbz], axis=1)
        diffT = rows - nbT
        dkT = jnp.sqrt(jnp.sum(diffT * diffT, axis=1, keepdims=True) + 1e-12)
        concatT = jnp.concatenate([rows, nbT, diffT, dkT], axis=1)
        concat = jnp.transpose(concatT)
        y1k = _mm(w1_ref[...], concat) + b1_ref[...]
        y3k = _mm(w2_ref[...], concat) + b2_ref[...]
        y1_ref[0, :, pl.ds(k, 1), :] = y1k[:, None, :]
        y3_ref[0, :, pl.ds(k, 1), :] = y3k[:, None, :]
        s1_ref[...] += _rowstats(y1k)
        s3_ref[...] += _rowstats(y3k)
        return Mv, Ml

    jax.lax.fori_loop(0, K, body, (Mv, Ml))


def _att_pool(x32, xf, c1_ref, c2_ref, s_ref, mw_ref, mb_ref):
    avg32 = jnp.mean(x32, axis=1)
    max32 = jnp.max(x32, axis=1)
    avg64 = jnp.concatenate([avg32, xf], axis=0)
    max64 = jnp.concatenate([max32, xf], axis=0)

    def cfc(t):
        h = jnp.maximum(_mm(c1_ref[...], t), 0.0)
        return _mm(c2_ref[...], h)

    ch = jax.nn.sigmoid(cfc(avg64) + cfc(max64))
    ch32 = ch[0:32][:, None, :]
    x32a = x32 * ch32
    xfa = xf * ch[32:64]

    a = (jnp.sum(x32a, axis=0) + jnp.sum(xfa, axis=0, keepdims=True)) / 64.0
    m = jnp.maximum(jnp.max(x32a, axis=0),
                    jnp.max(xfa, axis=0, keepdims=True))
    s00 = s_ref[0:1, 0:1]
    s01 = s_ref[0:1, 1:2]
    sp = jax.nn.sigmoid(s00 * a + s01 * m)
    fs32 = jnp.sum(x32a * sp[None, :, :], axis=1)
    fsf = xfa * jnp.sum(sp, axis=0, keepdims=True)
    fs = jnp.concatenate([fs32, fsf], axis=0)
    return _mm(mw_ref[...], fs) + mb_ref[...]


def _k2_body(y1_ref, s1_ref, f_ref, m1w_ref, m1b_ref, g1_ref, bt1_ref,
             c1_ref, c2_ref, s_ref, mw_ref, mb_ref, scw_ref, scb_ref,
             y2_ref, s2_ref, ssc_ref):
    bidx = pl.program_id(0)
    ridx = pl.program_id(1)

    @pl.when(jnp.logical_and(bidx == 0, ridx == 0))
    def _():
        s2_ref[...] = jnp.zeros_like(s2_ref)
        ssc_ref[...] = jnp.zeros_like(ssc_ref)

    scale, shift = _bn_affine(s1_ref, g1_ref, bt1_ref, CNT_LSE)
    x32 = jnp.maximum(scale[:, :, None] * y1_ref[0] + shift[:, :, None], 0.0)

    f = f_ref[0]
    xf0 = _mm(m1w_ref[...], f) + m1b_ref[...]
    xf = jnp.where(xf0 >= 0.0, xf0, 0.2 * xf0)

    y2 = _att_pool(x32, xf, c1_ref, c2_ref, s_ref, mw_ref, mb_ref)
    y2_ref[0] = y2
    s2_ref[...] += _rowstats(y2)

    shc = _mm(scw_ref[...], f) + scb_ref[...]
    ssc_ref[...] += _rowstats(shc)


def _k3_body(y3_ref, s3_ref, y2_ref, s2_ref, mg1_ref, mbt1_ref,
             g2_ref, bt2_ref, c1_ref, c2_ref, s_ref, mw_ref, mb_ref,
             y4_ref, s4_ref):
    bidx = pl.program_id(0)
    ridx = pl.program_id(1)

    @pl.when(jnp.logical_and(bidx == 0, ridx == 0))
    def _():
        s4_ref[...] = jnp.zeros_like(s4_ref)

    scale2, shift2 = _bn_affine(s2_ref, mg1_ref, mbt1_ref, CNT_PT)
    x2 = jnp.maximum(scale2 * y2_ref[0] + shift2, 0.0)

    scale3, shift3 = _bn_affine(s3_ref, g2_ref, bt2_ref, CNT_LSE)
    x32 = jnp.maximum(scale3[:, :, None] * y3_ref[0] + shift3[:, :, None],
                      0.0)

    y4 = _att_pool(x32, x2, c1_ref, c2_ref, s_ref, mw_ref, mb_ref)
    y4_ref[0] = y4
    s4_ref[...] += _rowstats(y4)


def _k4_body(y4_ref, s4_ref, mg2_ref, mbt2_ref, m2w_ref, m2b_ref,
             f_ref, scw_ref, scb_ref, ssc_ref, scg_ref, scbt_ref, o_ref):
    scale4, shift4 = _bn_affine(s4_ref, mg2_ref, mbt2_ref, CNT_PT)
    x3 = jnp.maximum(scale4 * y4_ref[0] + shift4, 0.0)
    main = _mm(m2w_ref[...], x3) + m2b_ref[...]

    f = f_ref[0]
    shc = _mm(scw_ref[...], f) + scb_ref[...]
    scs, scsh = _bn_affine(ssc_ref, scg_ref, scbt_ref, CNT_PT)
    o = main + (scs * shc + scsh)
    o_ref[0] = jnp.where(o >= 0.0, o, 0.01 * o)


def kernel(coords, features, mlp1_w, mlp1_b, lse1_w, lse1_b, lse1_g,
           lse1_bt, lse2_w, lse2_b, lse2_g, lse2_bt, p1_c1, p1_c2, p1_s,
           p1_mw, p1_mb, p1_mg, p1_mbt, p2_c1, p2_c2, p2_s, p2_mw, p2_mb,
           p2_mg, p2_mbt, mlp2_w, mlp2_b, sc_w, sc_b, sc_g, sc_bt):
    f32 = jnp.float32
    coordsT = jnp.transpose(coords, (0, 2, 1))
    coordsP = jnp.transpose(coords.reshape(B, NC, CW, 3),
                            (0, 1, 3, 2)).reshape(B, NC, 3 * CW)
    fR = features[:, :, :, 0]
    col = lambda v: v.reshape(-1, 1)

    y1, y3, s1, s3 = pl.pallas_call(
        _k1_body,
        grid=(B, NB1),
        in_specs=[
            pl.BlockSpec((1, 3, N), lambda b, r: (b, 0, 0)),
            pl.BlockSpec((1, R1, 3), lambda b, r: (b, r, 0)),
            pl.BlockSpec((1, NC, 3 * CW), lambda b, r: (b, 0, 0)),
            pl.BlockSpec((32, 10), lambda b, r: (0, 0)),
            pl.BlockSpec((32, 1), lambda b, r: (0, 0)),
            pl.BlockSpec((32, 10), lambda b, r: (0, 0)),
            pl.BlockSpec((32, 1), lambda b, r: (0, 0)),
        ],
        out_specs=[
            pl.BlockSpec((1, 32, K, R1), lambda b, r: (b, 0, 0, r)),
            pl.BlockSpec((1, 32, K, R1), lambda b, r: (b, 0, 0, r)),
            pl.BlockSpec((32, 2), lambda b, r: (0, 0)),
            pl.BlockSpec((32, 2), lambda b, r: (0, 0)),
        ],
        out_shape=[
            jax.ShapeDtypeStruct((B, 32, K, N), f32),
            jax.ShapeDtypeStruct((B, 32, K, N), f32),
            jax.ShapeDtypeStruct((32, 2), f32),
            jax.ShapeDtypeStruct((32, 2), f32),
        ],
        scratch_shapes=[pltpu.VMEM((R1, N), f32)],
    )(coordsT, coords, coordsP, lse1_w, col(lse1_b), lse2_w, col(lse2_b))

    y2, s2, ssc = pl.pallas_call(
        _k2_body,
        grid=(B, NB2),
        in_specs=[
            pl.BlockSpec((1, 32, K, R2), lambda b, r: (b, 0, 0, r)),
            pl.BlockSpec((32, 2), lambda b, r: (0, 0)),
            pl.BlockSpec((1, 8, R2), lambda b, r: (b, 0, r)),
            pl.BlockSpec((32, 8), lambda b, r: (0, 0)),
            pl.BlockSpec((32, 1), lambda b, r: (0, 0)),
            pl.BlockSpec((32, 1), lambda b, r: (0, 0)),
            pl.BlockSpec((32, 1), lambda b, r: (0, 0)),
            pl.BlockSpec((8, 64), lambda b, r: (0, 0)),
            pl.BlockSpec((64, 8), lambda b, r: (0, 0)),
            pl.BlockSpec((1, 2), lambda b, r: (0, 0)),
            pl.BlockSpec((32, 64), lambda b, r: (0, 0)),
            pl.BlockSpec((32, 1), lambda b, r: (0, 0)),
            pl.BlockSpec((128, 8), lambda b, r: (0, 0)),
            pl.BlockSpec((128, 1), lambda b, r: (0, 0)),
        ],
        out_specs=[
            pl.BlockSpec((1, 32, R2), lambda b, r: (b, 0, r)),
            pl.BlockSpec((32, 2), lambda b, r: (0, 0)),
            pl.BlockSpec((128, 2), lambda b, r: (0, 0)),
        ],
        out_shape=[
            jax.ShapeDtypeStruct((B, 32, N), f32),
            jax.ShapeDtypeStruct((32, 2), f32),
            jax.ShapeDtypeStruct((128, 2), f32),
        ],
    )(y1, s1, fR, mlp1_w, col(mlp1_b), col(lse1_g), col(lse1_bt),
      p1_c1, p1_c2, p1_s, p1_mw, col(p1_mb), sc_w, col(sc_b))

    y4, s4 = pl.pallas_call(
        _k3_body,
        grid=(B, NB2),
        in_specs=[
            pl.BlockSpec((1, 32, K, R2), lambda b, r: (b, 0, 0, r)),
            pl.BlockSpec((32, 2), lambda b, r: (0, 0)),
            pl.BlockSpec((1, 32, R2), lambda b, r: (b, 0, r)),
            pl.BlockSpec((32, 2), lambda b, r: (0, 0)),
            pl.BlockSpec((32, 1), lambda b, r: (0, 0)),
            pl.BlockSpec((32, 1), lambda b, r: (0, 0)),
            pl.BlockSpec((32, 1), lambda b, r: (0, 0)),
            pl.BlockSpec((32, 1), lambda b, r: (0, 0)),
            pl.BlockSpec((8, 64), lambda b, r: (0, 0)),
            pl.BlockSpec((64, 8), lambda b, r: (0, 0)),
            pl.BlockSpec((1, 2), lambda b, r: (0, 0)),
            pl.BlockSpec((64, 64), lambda b, r: (0, 0)),
            pl.BlockSpec((64, 1), lambda b, r: (0, 0)),
        ],
        out_specs=[
            pl.BlockSpec((1, 64, R2), lambda b, r: (b, 0, r)),
            pl.BlockSpec((64, 2), lambda b, r: (0, 0)),
        ],
        out_shape=[
            jax.ShapeDtypeStruct((B, 64, N), f32),
            jax.ShapeDtypeStruct((64, 2), f32),
        ],
    )(y3, s3, y2, s2, col(p1_mg), col(p1_mbt), col(lse2_g), col(lse2_bt),
      p2_c1, p2_c2, p2_s, p2_mw, col(p2_mb))

    out = pl.pallas_call(
        _k4_body,
        grid=(B, NB2),
        in_specs=[
            pl.BlockSpec((1, 64, R2), lambda b, r: (b, 0, r)),
            pl.BlockSpec((64, 2), lambda b, r: (0, 0)),
            pl.BlockSpec((64, 1), lambda b, r: (0, 0)),
            pl.BlockSpec((64, 1), lambda b, r: (0, 0)),
            pl.BlockSpec((128, 64), lambda b, r: (0, 0)),
            pl.BlockSpec((128, 1), lambda b, r: (0, 0)),
            pl.BlockSpec((1, 8, R2), lambda b, r: (b, 0, r)),
            pl.BlockSpec((128, 8), lambda b, r: (0, 0)),
            pl.BlockSpec((128, 1), lambda b, r: (0, 0)),
            pl.BlockSpec((128, 2), lambda b, r: (0, 0)),
            pl.BlockSpec((128, 1), lambda b, r: (0, 0)),
            pl.BlockSpec((128, 1), lambda b, r: (0, 0)),
        ],
        out_specs=[pl.BlockSpec((1, 128, R2), lambda b, r: (b, 0, r))],
        out_shape=[jax.ShapeDtypeStruct((B, 128, N), f32)],
    )(y4, s4, col(p2_mg), col(p2_mbt), mlp2_w, col(mlp2_b), fR,
      sc_w, col(sc_b), ssc, col(sc_g), col(sc_bt))[0]

    return out[:, :, :, None]


# speedup vs baseline: 1.5315x; 1.0132x over previous
import jax
import jax.numpy as jnp
from jax.experimental import pallas as pl
from jax.experimental.pallas import tpu as pltpu

B, N, K = 2, 4096, 16
R1 = 1024
R2 = 512
NB1 = N // R1
NB2 = N // R2
CNT_LSE = float(B * N * K)
CNT_PT = float(B * N)
EPS = 1e-5
BIG = 3.0e38


def _mm(a, b):
    return jax.lax.dot_general(
        a.astype(jnp.bfloat16), b.astype(jnp.bfloat16),
        (((a.ndim - 1,), (0,)), ((), ())),
        preferred_element_type=jnp.float32)


def _mm_exact(a, b):
    return jax.lax.dot_general(a, b, (((a.ndim - 1,), (0,)), ((), ())),
                               preferred_element_type=jnp.float32,
                               precision=jax.lax.Precision.HIGHEST)


def _bn_affine(sums_ref, g_ref, bt_ref, cnt):
    s = sums_ref[:, 0:1] / cnt
    q = sums_ref[:, 1:2] / cnt
    var = q - s * s
    scale = g_ref[...] * jax.lax.rsqrt(var + EPS)
    shift = bt_ref[...] - scale * s
    return scale, shift


def _rowstats(y):
    return jnp.concatenate(
        [jnp.sum(y, axis=1, keepdims=True),
         jnp.sum(y * y, axis=1, keepdims=True)], axis=1)


CW = 128
NC = N // CW


def _k1_body(ctf_ref, rows_ref, cpl_ref, w1_ref, b1_ref,
             w2_ref, b2_ref, y1_ref, y3_ref, s1_ref, s3_ref, dist_ref):
    bidx = pl.program_id(0)
    ridx = pl.program_id(1)

    @pl.when(jnp.logical_and(bidx == 0, ridx == 0))
    def _():
        s1_ref[...] = jnp.zeros_like(s1_ref)
        s3_ref[...] = jnp.zeros_like(s3_ref)

    ct = ctf_ref[0]
    rows = rows_ref[0]
    planes = cpl_ref[0]

    d2a = jnp.sum(ct * ct, axis=0, keepdims=True)
    g = _mm(rows, ct)
    score = d2a - 2.0 * g
    dist_ref[...] = score

    laneF = jax.lax.broadcasted_iota(jnp.int32, (R1, CW), 1).astype(
        jnp.float32)
    ciotaF = jax.lax.broadcasted_iota(jnp.int32, (R1, NC), 1).astype(
        jnp.float32)

    Mv = jnp.concatenate(
        [jnp.min(score[:, c * CW:(c + 1) * CW], axis=1, keepdims=True)
         for c in range(NC)], axis=1)
    Ml = jnp.full((R1, NC), -1.0, jnp.float32)

    def body(k, carry):
        Mv, Ml = carry
        vmin = jnp.min(Mv, axis=1, keepdims=True)
        cstar = jnp.min(jnp.where(Mv <= vmin, ciotaF, float(NC)),
                        axis=1, keepdims=True)
        ohb = ciotaF == cstar
        ohf = ohb.astype(jnp.float32)

        vals = dist_ref[:, 0:CW] * ohf[:, 0:1]
        for c in range(1, NC):
            vals = vals + dist_ref[:, c * CW:(c + 1) * CW] * ohf[:, c:c + 1]

        lraw = jnp.min(jnp.where(ohb, Ml, float(CW)), axis=1, keepdims=True)
        lfresh = jnp.min(jnp.where(vals <= vmin, laneF, float(CW)),
                         axis=1, keepdims=True)
        lstar = jnp.where(lraw < 0.0, lfresh, lraw)

        extracted = jnp.logical_or(
            vals < vmin, jnp.logical_and(vals == vmin, laneF <= lstar))
        masked = jnp.where(extracted, BIG, vals)
        newv = jnp.min(masked, axis=1, keepdims=True)
        newl = jnp.min(jnp.where(masked <= newv, laneF, float(CW)),
                       axis=1, keepdims=True)
        Mv = jnp.where(ohb, newv, Mv)
        Ml = jnp.where(ohb, newl, Ml)

        sel = _mm_exact(ohf, planes)
        lm = (laneF == lstar).astype(jnp.float32)
        nbx = jnp.sum(sel[:, 0:CW] * lm, axis=1, keepdims=True)
        nby = jnp.sum(sel[:, CW:2 * CW] * lm, axis=1, keepdims=True)
        nbz = jnp.sum(sel[:, 2 * CW:3 * CW] * lm, axis=1, keepdims=True)
        nbT = jnp.concatenate([nbx, nby, nbz], axis=1)
        diffT = rows - nbT
        dkT = jnp.sqrt(jnp.sum(diffT * diffT, axis=1, keepdims=True) + 1e-12)
        concatT = jnp.concatenate([rows, nbT, diffT, dkT], axis=1)
        concat = jnp.transpose(concatT)
        y1k = _mm(w1_ref[...], concat) + b1_ref[...]
        y3k = _mm(w2_ref[...], concat) + b2_ref[...]
        y1_ref[0, :, pl.ds(k, 1), :] = y1k[:, None, :]
        y3_ref[0, :, pl.ds(k, 1), :] = y3k[:, None, :]
        s1_ref[...] += _rowstats(y1k)
        s3_ref[...] += _rowstats(y3k)
        return Mv, Ml

    jax.lax.fori_loop(0, K, body, (Mv, Ml))


def _att_pool(x32, xf, c1_ref, c2_ref, s_ref, mw_ref, mb_ref):
    avg32 = jnp.mean(x32, axis=1)
    max32 = jnp.max(x32, axis=1)
    avg64 = jnp.concatenate([avg32, xf], axis=0)
    max64 = jnp.concatenate([max32, xf], axis=0)

    def cfc(t):
        h = jnp.maximum(_mm(c1_ref[...], t), 0.0)
        return _mm(c2_ref[...], h)

    ch = jax.nn.sigmoid(cfc(avg64) + cfc(max64))
    ch32 = ch[0:32][:, None, :]
    x32a = x32 * ch32
    xfa = xf * ch[32:64]

    a = (jnp.sum(x32a, axis=0) + jnp.sum(xfa, axis=0, keepdims=True)) / 64.0
    m = jnp.maximum(jnp.max(x32a, axis=0),
                    jnp.max(xfa, axis=0, keepdims=True))
    s00 = s_ref[0:1, 0:1]
    s01 = s_ref[0:1, 1:2]
    sp = jax.nn.sigmoid(s00 * a + s01 * m)
    fs32 = jnp.sum(x32a * sp[None, :, :], axis=1)
    fsf = xfa * jnp.sum(sp, axis=0, keepdims=True)
    fs = jnp.concatenate([fs32, fsf], axis=0)
    return _mm(mw_ref[...], fs) + mb_ref[...]


def _k2_body(y1_ref, s1_ref, f_ref, m1w_ref, m1b_ref, g1_ref, bt1_ref,
             c1_ref, c2_ref, s_ref, mw_ref, mb_ref, scw_ref, scb_ref,
             y2_ref, s2_ref, ssc_ref):
    bidx = pl.program_id(0)
    ridx = pl.program_id(1)

    @pl.when(jnp.logical_and(bidx == 0, ridx == 0))
    def _():
        s2_ref[...] = jnp.zeros_like(s2_ref)
        ssc_ref[...] = jnp.zeros_like(ssc_ref)

    scale, shift = _bn_affine(s1_ref, g1_ref, bt1_ref, CNT_LSE)
    x32 = jnp.maximum(scale[:, :, None] * y1_ref[0] + shift[:, :, None], 0.0)

    f = f_ref[0]
    xf0 = _mm(m1w_ref[...], f) + m1b_ref[...]
    xf = jnp.where(xf0 >= 0.0, xf0, 0.2 * xf0)

    y2 = _att_pool(x32, xf, c1_ref, c2_ref, s_ref, mw_ref, mb_ref)
    y2_ref[0] = y2
    s2_ref[...] += _rowstats(y2)

    shc = _mm(scw_ref[...], f) + scb_ref[...]
    ssc_ref[...] += _rowstats(shc)


def _k3_body(y3_ref, s3_ref, y2_ref, s2_ref, mg1_ref, mbt1_ref,
             g2_ref, bt2_ref, c1_ref, c2_ref, s_ref, mw_ref, mb_ref,
             y4_ref, s4_ref):
    bidx = pl.program_id(0)
    ridx = pl.program_id(1)

    @pl.when(jnp.logical_and(bidx == 0, ridx == 0))
    def _():
        s4_ref[...] = jnp.zeros_like(s4_ref)

    scale2, shift2 = _bn_affine(s2_ref, mg1_ref, mbt1_ref, CNT_PT)
    x2 = jnp.maximum(scale2 * y2_ref[0] + shift2, 0.0)

    scale3, shift3 = _bn_affine(s3_ref, g2_ref, bt2_ref, CNT_LSE)
    x32 = jnp.maximum(scale3[:, :, None] * y3_ref[0] + shift3[:, :, None],
                      0.0)

    y4 = _att_pool(x32, x2, c1_ref, c2_ref, s_ref, mw_ref, mb_ref)
    y4_ref[0] = y4
    s4_ref[...] += _rowstats(y4)


def _k4_body(y4_ref, s4_ref, mg2_ref, mbt2_ref, m2w_ref, m2b_ref,
             f_ref, scw_ref, scb_ref, ssc_ref, scg_ref, scbt_ref, o_ref):
    scale4, shift4 = _bn_affine(s4_ref, mg2_ref, mbt2_ref, CNT_PT)
    x3 = jnp.maximum(scale4 * y4_ref[0] + shift4, 0.0)
    main = _mm(m2w_ref[...], x3) + m2b_ref[...]

    f = f_ref[0]
    shc = _mm(scw_ref[...], f) + scb_ref[...]
    scs, scsh = _bn_affine(ssc_ref, scg_ref, scbt_ref, CNT_PT)
    o = main + (scs * shc + scsh)
    o_ref[0] = jnp.where(o >= 0.0, o, 0.01 * o)


def kernel(coords, features, mlp1_w, mlp1_b, lse1_w, lse1_b, lse1_g,
           lse1_bt, lse2_w, lse2_b, lse2_g, lse2_bt, p1_c1, p1_c2, p1_s,
           p1_mw, p1_mb, p1_mg, p1_mbt, p2_c1, p2_c2, p2_s, p2_mw, p2_mb,
           p2_mg, p2_mbt, mlp2_w, mlp2_b, sc_w, sc_b, sc_g, sc_bt):
    f32 = jnp.float32
    coordsT = jnp.transpose(coords, (0, 2, 1))
    coordsP = jnp.transpose(coords.reshape(B, NC, CW, 3),
                            (0, 1, 3, 2)).reshape(B, NC, 3 * CW)
    fR = features[:, :, :, 0]
    col = lambda v: v.reshape(-1, 1)

    y1, y3, s1, s3 = pl.pallas_call(
        _k1_body,
        grid=(B, NB1),
        in_specs=[
            pl.BlockSpec((1, 3, N), lambda b, r: (b, 0, 0)),
            pl.BlockSpec((1, R1, 3), lambda b, r: (b, r, 0)),
            pl.BlockSpec((1, NC, 3 * CW), lambda b, r: (b, 0, 0)),
            pl.BlockSpec((32, 10), lambda b, r: (0, 0)),
            pl.BlockSpec((32, 1), lambda b, r: (0, 0)),
            pl.BlockSpec((32, 10), lambda b, r: (0, 0)),
            pl.BlockSpec((32, 1), lambda b, r: (0, 0)),
        ],
        out_specs=[
            pl.BlockSpec((1, 32, K, R1), lambda b, r: (b, 0, 0, r)),
            pl.BlockSpec((1, 32, K, R1), lambda b, r: (b, 0, 0, r)),
            pl.BlockSpec((32, 2), lambda b, r: (0, 0)),
            pl.BlockSpec((32, 2), lambda b, r: (0, 0)),
        ],
        out_shape=[
            jax.ShapeDtypeStruct((B, 32, K, N), f32),
            jax.ShapeDtypeStruct((B, 32, K, N), f32),
            jax.ShapeDtypeStruct((32, 2), f32),
            jax.ShapeDtypeStruct((32, 2), f32),
        ],
        scratch_shapes=[pltpu.VMEM((R1, N), f32)],
    )(coordsT, coords, coordsP, lse1_w, col(lse1_b), lse2_w, col(lse2_b))

    y2, s2, ssc = pl.pallas_call(
        _k2_body,
        grid=(B, NB2),
        in_specs=[
            pl.BlockSpec((1, 32, K, R2), lambda b, r: (b, 0, 0, r)),
            pl.BlockSpec((32, 2), lambda b, r: (0, 0)),
            pl.BlockSpec((1, 8, R2), lambda b, r: (b, 0, r)),
            pl.BlockSpec((32, 8), lambda b, r: (0, 0)),
            pl.BlockSpec((32, 1), lambda b, r: (0, 0)),
            pl.BlockSpec((32, 1), lambda b, r: (0, 0)),
            pl.BlockSpec((32, 1), lambda b, r: (0, 0)),
            pl.BlockSpec((8, 64), lambda b, r: (0, 0)),
            pl.BlockSpec((64, 8), lambda b, r: (0, 0)),
            pl.BlockSpec((1, 2), lambda b, r: (0, 0)),
            pl.BlockSpec((32, 64), lambda b, r: (0, 0)),
            pl.BlockSpec((32, 1), lambda b, r: (0, 0)),
            pl.BlockSpec((128, 8), lambda b, r: (0, 0)),
            pl.BlockSpec((128, 1), lambda b, r: (0, 0)),
        ],
        out_specs=[
            pl.BlockSpec((1, 32, R2), lambda b, r: (b, 0, r)),
            pl.BlockSpec((32, 2), lambda b, r: (0, 0)),
            pl.BlockSpec((128, 2), lambda b, r: (0, 0)),
        ],
        out_shape=[
            jax.ShapeDtypeStruct((B, 32, N), f32),
            jax.ShapeDtypeStruct((32, 2), f32),
            jax.ShapeDtypeStruct((128, 2), f32),
        ],
    )(y1, s1, fR, mlp1_w, col(mlp1_b), col(lse1_g), col(lse1_bt),
      p1_c1, p1_c2, p1_s, p1_mw, col(p1_mb), sc_w, col(sc_b))

    y4, s4 = pl.pallas_call(
        _k3_body,
        grid=(B, NB2),
        in_specs=[
            pl.BlockSpec((1, 32, K, R2), lambda b, r: (b, 0, 0, r)),
            pl.BlockSpec((32, 2), lambda b, r: (0, 0)),
            pl.BlockSpec((1, 32, R2), lambda b, r: (b, 0, r)),
            pl.BlockSpec((32, 2), lambda b, r: (0, 0)),
            pl.BlockSpec((32, 1), lambda b, r: (0, 0)),
            pl.BlockSpec((32, 1), lambda b, r: (0, 0)),
            pl.BlockSpec((32, 1), lambda b, r: (0, 0)),
            pl.BlockSpec((32, 1), lambda b, r: (0, 0)),
            pl.BlockSpec((8, 64), lambda b, r: (0, 0)),
            pl.BlockSpec((64, 8), lambda b, r: (0, 0)),
            pl.BlockSpec((1, 2), lambda b, r: (0, 0)),
            pl.BlockSpec((64, 64), lambda b, r: (0, 0)),
            pl.BlockSpec((64, 1), lambda b, r: (0, 0)),
        ],
        out_specs=[
            pl.BlockSpec((1, 64, R2), lambda b, r: (b, 0, r)),
            pl.BlockSpec((64, 2), lambda b, r: (0, 0)),
        ],
        out_shape=[
            jax.ShapeDtypeStruct((B, 64, N), f32),
            jax.ShapeDtypeStruct((64, 2), f32),
        ],
    )(y3, s3, y2, s2, col(p1_mg), col(p1_mbt), col(lse2_g), col(lse2_bt),
      p2_c1, p2_c2, p2_s, p2_mw, col(p2_mb))

    out = pl.pallas_call(
        _k4_body,
        grid=(B, NB2),
        in_specs=[
            pl.BlockSpec((1, 64, R2), lambda b, r: (b, 0, r)),
            pl.BlockSpec((64, 2), lambda b, r: (0, 0)),
            pl.BlockSpec((64, 1), lambda b, r: (0, 0)),
            pl.BlockSpec((64, 1), lambda b, r: (0, 0)),
            pl.BlockSpec((128, 64), lambda b, r: (0, 0)),
            pl.BlockSpec((128, 1), lambda b, r: (0, 0)),
            pl.BlockSpec((1, 8, R2), lambda b, r: (b, 0, r)),
            pl.BlockSpec((128, 8), lambda b, r: (0, 0)),
            pl.BlockSpec((128, 1), lambda b, r: (0, 0)),
            pl.BlockSpec((128, 2), lambda b, r: (0, 0)),
            pl.BlockSpec((128, 1), lambda b, r: (0, 0)),
            pl.BlockSpec((128, 1), lambda b, r: (0, 0)),
        ],
        out_specs=[pl.BlockSpec((1, 128, R2), lambda b, r: (b, 0, r))],
        out_shape=[jax.ShapeDtypeStruct((B, 128, N), f32)],
    )(y4, s4, col(p2_mg), col(p2_mbt), mlp2_w, col(mlp2_b), fR,
      sc_w, col(sc_b), ssc, col(sc_g), col(sc_bt))[0]

    return out[:, :, :, None]


# speedup vs baseline: 1.5605x; 1.0190x over previous
import jax
import jax.numpy as jnp
from jax.experimental import pallas as pl
from jax.experimental.pallas import tpu as pltpu

B, N, K = 2, 4096, 16
R1 = 1024
R2 = 1024
NB1 = N // R1
NB2 = N // R2
CNT_LSE = float(B * N * K)
CNT_PT = float(B * N)
EPS = 1e-5
BIG = 3.0e38


def _mm(a, b):
    return jax.lax.dot_general(
        a.astype(jnp.bfloat16), b.astype(jnp.bfloat16),
        (((a.ndim - 1,), (0,)), ((), ())),
        preferred_element_type=jnp.float32)


def _mm_exact(a, b):
    return jax.lax.dot_general(a, b, (((a.ndim - 1,), (0,)), ((), ())),
                               preferred_element_type=jnp.float32,
                               precision=jax.lax.Precision.HIGHEST)


def _bn_affine(sums_ref, g_ref, bt_ref, cnt):
    s = sums_ref[:, 0:1] / cnt
    q = sums_ref[:, 1:2] / cnt
    var = q - s * s
    scale = g_ref[...] * jax.lax.rsqrt(var + EPS)
    shift = bt_ref[...] - scale * s
    return scale, shift


def _rowstats(y):
    return jnp.concatenate(
        [jnp.sum(y, axis=1, keepdims=True),
         jnp.sum(y * y, axis=1, keepdims=True)], axis=1)


CW = 128
NC = N // CW


def _k1_body(ctf_ref, rows_ref, cpl_ref, w1_ref, b1_ref,
             w2_ref, b2_ref, y1_ref, y3_ref, s1_ref, s3_ref, dist_ref):
    bidx = pl.program_id(0)
    ridx = pl.program_id(1)

    @pl.when(jnp.logical_and(bidx == 0, ridx == 0))
    def _():
        s1_ref[...] = jnp.zeros_like(s1_ref)
        s3_ref[...] = jnp.zeros_like(s3_ref)

    ct = ctf_ref[0]
    rows = rows_ref[0]
    planes = cpl_ref[0]

    d2a = jnp.sum(ct * ct, axis=0, keepdims=True)
    g = _mm(rows, ct)
    score = d2a - 2.0 * g
    dist_ref[...] = score

    laneF = jax.lax.broadcasted_iota(jnp.int32, (R1, CW), 1).astype(
        jnp.float32)
    ciotaF = jax.lax.broadcasted_iota(jnp.int32, (R1, NC), 1).astype(
        jnp.float32)

    Mv = jnp.concatenate(
        [jnp.min(score[:, c * CW:(c + 1) * CW], axis=1, keepdims=True)
         for c in range(NC)], axis=1)
    Ml = jnp.full((R1, NC), -1.0, jnp.float32)

    def body(k, carry):
        Mv, Ml = carry
        vmin = jnp.min(Mv, axis=1, keepdims=True)
        cstar = jnp.min(jnp.where(Mv <= vmin, ciotaF, float(NC)),
                        axis=1, keepdims=True)
        ohb = ciotaF == cstar
        ohf = ohb.astype(jnp.float32)

        vals = dist_ref[:, 0:CW] * ohf[:, 0:1]
        for c in range(1, NC):
            vals = vals + dist_ref[:, c * CW:(c + 1) * CW] * ohf[:, c:c + 1]

        lraw = jnp.min(jnp.where(ohb, Ml, float(CW)), axis=1, keepdims=True)
        lfresh = jnp.min(jnp.where(vals <= vmin, laneF, float(CW)),
                         axis=1, keepdims=True)
        lstar = jnp.where(lraw < 0.0, lfresh, lraw)

        extracted = jnp.logical_or(
            vals < vmin, jnp.logical_and(vals == vmin, laneF <= lstar))
        masked = jnp.where(extracted, BIG, vals)
        newv = jnp.min(masked, axis=1, keepdims=True)
        newl = jnp.min(jnp.where(masked <= newv, laneF, float(CW)),
                       axis=1, keepdims=True)
        Mv = jnp.where(ohb, newv, Mv)
        Ml = jnp.where(ohb, newl, Ml)

        sel = _mm_exact(ohf, planes)
        lm = (laneF == lstar).astype(jnp.float32)
        nbx = jnp.sum(sel[:, 0:CW] * lm, axis=1, keepdims=True)
        nby = jnp.sum(sel[:, CW:2 * CW] * lm, axis=1, keepdims=True)
        nbz = jnp.sum(sel[:, 2 * CW:3 * CW] * lm, axis=1, keepdims=True)
        nbT = jnp.concatenate([nbx, nby, nbz], axis=1)
        diffT = rows - nbT
        dkT = jnp.sqrt(jnp.sum(diffT * diffT, axis=1, keepdims=True) + 1e-12)
        concatT = jnp.concatenate([rows, nbT, diffT, dkT], axis=1)
        concat = jnp.transpose(concatT)
        y1k = _mm(w1_ref[...], concat) + b1_ref[...]
        y3k = _mm(w2_ref[...], concat) + b2_ref[...]
        y1_ref[0, :, pl.ds(k, 1), :] = y1k[:, None, :]
        y3_ref[0, :, pl.ds(k, 1), :] = y3k[:, None, :]
        s1_ref[...] += _rowstats(y1k)
        s3_ref[...] += _rowstats(y3k)
        return Mv, Ml

    jax.lax.fori_loop(0, K, body, (Mv, Ml))


def _att_pool(x32, xf, c1_ref, c2_ref, s_ref, mw_ref, mb_ref):
    avg32 = jnp.mean(x32, axis=1)
    max32 = jnp.max(x32, axis=1)
    avg64 = jnp.concatenate([avg32, xf], axis=0)
    max64 = jnp.concatenate([max32, xf], axis=0)

    def cfc(t):
        h = jnp.maximum(_mm(c1_ref[...], t), 0.0)
        return _mm(c2_ref[...], h)

    ch = jax.nn.sigmoid(cfc(avg64) + cfc(max64))
    ch32 = ch[0:32][:, None, :]
    x32a = x32 * ch32
    xfa = xf * ch[32:64]

    a = (jnp.sum(x32a, axis=0) + jnp.sum(xfa, axis=0, keepdims=True)) / 64.0
    m = jnp.maximum(jnp.max(x32a, axis=0),
                    jnp.max(xfa, axis=0, keepdims=True))
    s00 = s_ref[0:1, 0:1]
    s01 = s_ref[0:1, 1:2]
    sp = jax.nn.sigmoid(s00 * a + s01 * m)
    fs32 = jnp.sum(x32a * sp[None, :, :], axis=1)
    fsf = xfa * jnp.sum(sp, axis=0, keepdims=True)
    fs = jnp.concatenate([fs32, fsf], axis=0)
    return _mm(mw_ref[...], fs) + mb_ref[...]


def _k2_body(y1_ref, s1_ref, f_ref, m1w_ref, m1b_ref, g1_ref, bt1_ref,
             c1_ref, c2_ref, s_ref, mw_ref, mb_ref, scw_ref, scb_ref,
             y2_ref, s2_ref, ssc_ref):
    bidx = pl.program_id(0)
    ridx = pl.program_id(1)

    @pl.when(jnp.logical_and(bidx == 0, ridx == 0))
    def _():
        s2_ref[...] = jnp.zeros_like(s2_ref)
        ssc_ref[...] = jnp.zeros_like(ssc_ref)

    scale, shift = _bn_affine(s1_ref, g1_ref, bt1_ref, CNT_LSE)
    x32 = jnp.maximum(scale[:, :, None] * y1_ref[0] + shift[:, :, None], 0.0)

    f = f_ref[0]
    xf0 = _mm(m1w_ref[...], f) + m1b_ref[...]
    xf = jnp.where(xf0 >= 0.0, xf0, 0.2 * xf0)

    y2 = _att_pool(x32, xf, c1_ref, c2_ref, s_ref, mw_ref, mb_ref)
    y2_ref[0] = y2
    s2_ref[...] += _rowstats(y2)

    shc = _mm(scw_ref[...], f) + scb_ref[...]
    ssc_ref[...] += _rowstats(shc)


def _k3_body(y3_ref, s3_ref, y2_ref, s2_ref, mg1_ref, mbt1_ref,
             g2_ref, bt2_ref, c1_ref, c2_ref, s_ref, mw_ref, mb_ref,
             y4_ref, s4_ref):
    bidx = pl.program_id(0)
    ridx = pl.program_id(1)

    @pl.when(jnp.logical_and(bidx == 0, ridx == 0))
    def _():
        s4_ref[...] = jnp.zeros_like(s4_ref)

    scale2, shift2 = _bn_affine(s2_ref, mg1_ref, mbt1_ref, CNT_PT)
    x2 = jnp.maximum(scale2 * y2_ref[0] + shift2, 0.0)

    scale3, shift3 = _bn_affine(s3_ref, g2_ref, bt2_ref, CNT_LSE)
    x32 = jnp.maximum(scale3[:, :, None] * y3_ref[0] + shift3[:, :, None],
                      0.0)

    y4 = _att_pool(x32, x2, c1_ref, c2_ref, s_ref, mw_ref, mb_ref)
    y4_ref[0] = y4
    s4_ref[...] += _rowstats(y4)


def _k4_body(y4_ref, s4_ref, mg2_ref, mbt2_ref, m2w_ref, m2b_ref,
             f_ref, scw_ref, scb_ref, ssc_ref, scg_ref, scbt_ref, o_ref):
    scale4, shift4 = _bn_affine(s4_ref, mg2_ref, mbt2_ref, CNT_PT)
    x3 = jnp.maximum(scale4 * y4_ref[0] + shift4, 0.0)
    main = _mm(m2w_ref[...], x3) + m2b_ref[...]

    f = f_ref[0]
    shc = _mm(scw_ref[...], f) + scb_ref[...]
    scs, scsh = _bn_affine(ssc_ref, scg_ref, scbt_ref, CNT_PT)
    o = main + (scs * shc + scsh)
    o_ref[0] = jnp.where(o >= 0.0, o, 0.01 * o)


def kernel(coords, features, mlp1_w, mlp1_b, lse1_w, lse1_b, lse1_g,
           lse1_bt, lse2_w, lse2_b, lse2_g, lse2_bt, p1_c1, p1_c2, p1_s,
           p1_mw, p1_mb, p1_mg, p1_mbt, p2_c1, p2_c2, p2_s, p2_mw, p2_mb,
           p2_mg, p2_mbt, mlp2_w, mlp2_b, sc_w, sc_b, sc_g, sc_bt):
    f32 = jnp.float32
    coordsT = jnp.transpose(coords, (0, 2, 1))
    coordsP = jnp.transpose(coords.reshape(B, NC, CW, 3),
                            (0, 1, 3, 2)).reshape(B, NC, 3 * CW)
    fR = features[:, :, :, 0]
    col = lambda v: v.reshape(-1, 1)

    y1, y3, s1, s3 = pl.pallas_call(
        _k1_body,
        grid=(B, NB1),
        in_specs=[
            pl.BlockSpec((1, 3, N), lambda b, r: (b, 0, 0)),
            pl.BlockSpec((1, R1, 3), lambda b, r: (b, r, 0)),
            pl.BlockSpec((1, NC, 3 * CW), lambda b, r: (b, 0, 0)),
            pl.BlockSpec((32, 10), lambda b, r: (0, 0)),
            pl.BlockSpec((32, 1), lambda b, r: (0, 0)),
            pl.BlockSpec((32, 10), lambda b, r: (0, 0)),
            pl.BlockSpec((32, 1), lambda b, r: (0, 0)),
        ],
        out_specs=[
            pl.BlockSpec((1, 32, K, R1), lambda b, r: (b, 0, 0, r)),
            pl.BlockSpec((1, 32, K, R1), lambda b, r: (b, 0, 0, r)),
            pl.BlockSpec((32, 2), lambda b, r: (0, 0)),
            pl.BlockSpec((32, 2), lambda b, r: (0, 0)),
        ],
        out_shape=[
            jax.ShapeDtypeStruct((B, 32, K, N), f32),
            jax.ShapeDtypeStruct((B, 32, K, N), f32),
            jax.ShapeDtypeStruct((32, 2), f32),
            jax.ShapeDtypeStruct((32, 2), f32),
        ],
        scratch_shapes=[pltpu.VMEM((R1, N), f32)],
    )(coordsT, coords, coordsP, lse1_w, col(lse1_b), lse2_w, col(lse2_b))

    y2, s2, ssc = pl.pallas_call(
        _k2_body,
        grid=(B, NB2),
        in_specs=[
            pl.BlockSpec((1, 32, K, R2), lambda b, r: (b, 0, 0, r)),
            pl.BlockSpec((32, 2), lambda b, r: (0, 0)),
            pl.BlockSpec((1, 8, R2), lambda b, r: (b, 0, r)),
            pl.BlockSpec((32, 8), lambda b, r: (0, 0)),
            pl.BlockSpec((32, 1), lambda b, r: (0, 0)),
            pl.BlockSpec((32, 1), lambda b, r: (0, 0)),
            pl.BlockSpec((32, 1), lambda b, r: (0, 0)),
            pl.BlockSpec((8, 64), lambda b, r: (0, 0)),
            pl.BlockSpec((64, 8), lambda b, r: (0, 0)),
            pl.BlockSpec((1, 2), lambda b, r: (0, 0)),
            pl.BlockSpec((32, 64), lambda b, r: (0, 0)),
            pl.BlockSpec((32, 1), lambda b, r: (0, 0)),
            pl.BlockSpec((128, 8), lambda b, r: (0, 0)),
            pl.BlockSpec((128, 1), lambda b, r: (0, 0)),
        ],
        out_specs=[
            pl.BlockSpec((1, 32, R2), lambda b, r: (b, 0, r)),
            pl.BlockSpec((32, 2), lambda b, r: (0, 0)),
            pl.BlockSpec((128, 2), lambda b, r: (0, 0)),
        ],
        out_shape=[
            jax.ShapeDtypeStruct((B, 32, N), f32),
            jax.ShapeDtypeStruct((32, 2), f32),
            jax.ShapeDtypeStruct((128, 2), f32),
        ],
    )(y1, s1, fR, mlp1_w, col(mlp1_b), col(lse1_g), col(lse1_bt),
      p1_c1, p1_c2, p1_s, p1_mw, col(p1_mb), sc_w, col(sc_b))

    y4, s4 = pl.pallas_call(
        _k3_body,
        grid=(B, NB2),
        in_specs=[
            pl.BlockSpec((1, 32, K, R2), lambda b, r: (b, 0, 0, r)),
            pl.BlockSpec((32, 2), lambda b, r: (0, 0)),
            pl.BlockSpec((1, 32, R2), lambda b, r: (b, 0, r)),
            pl.BlockSpec((32, 2), lambda b, r: (0, 0)),
            pl.BlockSpec((32, 1), lambda b, r: (0, 0)),
            pl.BlockSpec((32, 1), lambda b, r: (0, 0)),
            pl.BlockSpec((32, 1), lambda b, r: (0, 0)),
            pl.BlockSpec((32, 1), lambda b, r: (0, 0)),
            pl.BlockSpec((8, 64), lambda b, r: (0, 0)),
            pl.BlockSpec((64, 8), lambda b, r: (0, 0)),
            pl.BlockSpec((1, 2), lambda b, r: (0, 0)),
            pl.BlockSpec((64, 64), lambda b, r: (0, 0)),
            pl.BlockSpec((64, 1), lambda b, r: (0, 0)),
        ],
        out_specs=[
            pl.BlockSpec((1, 64, R2), lambda b, r: (b, 0, r)),
            pl.BlockSpec((64, 2), lambda b, r: (0, 0)),
        ],
        out_shape=[
            jax.ShapeDtypeStruct((B, 64, N), f32),
            jax.ShapeDtypeStruct((64, 2), f32),
        ],
    )(y3, s3, y2, s2, col(p1_mg), col(p1_mbt), col(lse2_g), col(lse2_bt),
      p2_c1, p2_c2, p2_s, p2_mw, col(p2_mb))

    out = pl.pallas_call(
        _k4_body,
        grid=(B, NB2),
        in_specs=[
            pl.BlockSpec((1, 64, R2), lambda b, r: (b, 0, r)),
            pl.BlockSpec((64, 2), lambda b, r: (0, 0)),
            pl.BlockSpec((64, 1), lambda b, r: (0, 0)),
            pl.BlockSpec((64, 1), lambda b, r: (0, 0)),
            pl.BlockSpec((128, 64), lambda b, r: (0, 0)),
            pl.BlockSpec((128, 1), lambda b, r: (0, 0)),
            pl.BlockSpec((1, 8, R2), lambda b, r: (b, 0, r)),
            pl.BlockSpec((128, 8), lambda b, r: (0, 0)),
            pl.BlockSpec((128, 1), lambda b, r: (0, 0)),
            pl.BlockSpec((128, 2), lambda b, r: (0, 0)),
            pl.BlockSpec((128, 1), lambda b, r: (0, 0)),
            pl.BlockSpec((128, 1), lambda b, r: (0, 0)),
        ],
        out_specs=[pl.BlockSpec((1, 128, R2), lambda b, r: (b, 0, r))],
        out_shape=[jax.ShapeDtypeStruct((B, 128, N), f32)],
    )(y4, s4, col(p2_mg), col(p2_mbt), mlp2_w, col(mlp2_b), fR,
      sc_w, col(sc_b), ssc, col(sc_g), col(sc_bt))[0]

    return out[:, :, :, None]
